# baseline (speedup 1.0000x reference)
import jax
import jax.numpy as jnp
from jax import lax
from jax.experimental import pallas as pl
from jax.experimental.pallas import tpu as pltpu
from jax.experimental.pallas import tpu_sc as plsc

_EPS = 1e-5
_GRID = 10
_NODE_ROWS = 10000
_EDGE_ROWS = 320000
_BL = _EDGE_ROWS // _GRID

_NW = 32
_CROWS = 200
_NCHUNKS = _NODE_ROWS // _CROWS
_JMAX = (_NCHUNKS + _NW - 1) // _NW


def _tc_body(e_ref, t_ref, m_ref, eg_ref, eb_ref, tg_ref, tb_ref,
             eo_ref, to_ref):
    row8 = jax.lax.broadcasted_iota(jnp.int32, (8, _BL), 0)
    geom8 = row8 < 3
    m8 = m_ref[...]
    for ref, g_ref, b_ref, o_ref in ((e_ref, eg_ref, eb_ref, eo_ref),
                                     (t_ref, tg_ref, tb_ref, to_ref)):
        x = ref[...]
        s1 = jnp.dot(m8, x, preferred_element_type=jnp.float32)
        s2 = jnp.dot(m8, x * x, preferred_element_type=jnp.float32)
        mu = s1 * (1.0 / 13.0)
        var = s2 * (1.0 / 13.0) - mu * mu
        rstd = jax.lax.rsqrt(var + _EPS)
        g = g_ref[...]
        b = b_ref[...]
        p_lo = rstd * g[0:8, :]
        p_hi = rstd * g[8:16, :]
        q_lo = b[0:8, :] - mu * p_lo
        q_hi = b[8:16, :] - mu * p_hi
        xlo = x[0:8, :]
        xhi = x[8:16, :]
        o_ref[0:8, :] = jnp.where(geom8, xlo, xlo * p_lo + q_lo)
        o_ref[8:16, :] = xhi * p_hi + q_hi


def _sc_rsqrt(v):
    iv = lax.bitcast_convert_type(v, jnp.int32)
    y = lax.bitcast_convert_type(jnp.int32(0x5F3759DF) - (iv >> 1), jnp.float32)
    for _ in range(3):
        y = y * (1.5 - 0.5 * v * y * y)
    return y


def _sc_node_body(n_ref, g_ref, b_ref, o_ref, xbuf, obuf, gbuf, bbuf):
    wid = lax.axis_index("s") * 2 + lax.axis_index("c")
    pltpu.sync_copy(g_ref, gbuf)
    pltpu.sync_copy(b_ref, bbuf)

    def chunk_body(j, carry):
        cid = wid + j * _NW

        @pl.when(cid < _NCHUNKS)
        def _():
            r0 = cid * _CROWS
            pltpu.sync_copy(n_ref.at[pl.ds(r0, _CROWS), :], xbuf)

            def row(k, c2):
                xs = [xbuf[k, pl.ds(l * 16, 16)] for l in range(8)]
                s1v = xs[0]
                for l in range(1, 8):
                    s1v = s1v + xs[l]
                s2v = xs[0] * xs[0]
                for l in range(1, 8):
                    s2v = s2v + xs[l] * xs[l]
                s1 = jnp.sum(s1v)
                s2 = jnp.sum(s2v)
                muv = jnp.full((16,), s1) * (1.0 / 128.0)
                s2m = jnp.full((16,), s2) * (1.0 / 128.0)
                var = s2m - muv * muv
                rstd = _sc_rsqrt(var + _EPS)
                for l in range(8):
                    obuf[k, pl.ds(l * 16, 16)] = ((xs[l] - muv) * rstd
                                                  * gbuf[l, :] + bbuf[l, :])
                return c2

            lax.fori_loop(0, _CROWS, row, 0, unroll=4)
            pltpu.sync_copy(obuf, o_ref.at[pl.ds(r0, _CROWS), :])

        return carry

    lax.fori_loop(0, _JMAX, chunk_body, 0)


def _col16(vec13):
    return jnp.concatenate([jnp.zeros((3,), vec13.dtype), vec13])[:, None]


def kernel(node_features, edge_features, triangle_features,
           node_gamma, node_beta, edge_gamma, edge_beta, tri_gamma, tri_beta):
    e_t = edge_features.T
    t_t = triangle_features.T
    m8 = (jnp.arange(16)[None, :] >= 3).astype(jnp.float32) * jnp.ones((8, 1), jnp.float32)

    mesh = plsc.VectorSubcoreMesh(core_axis_name="c", subcore_axis_name="s")
    node_out = pl.kernel(
        _sc_node_body,
        out_type=jax.ShapeDtypeStruct((_NODE_ROWS, 128), jnp.float32),
        mesh=mesh,
        scratch_types=[
            pltpu.VMEM((_CROWS, 128), jnp.float32),
            pltpu.VMEM((_CROWS, 128), jnp.float32),
            pltpu.VMEM((8, 16), jnp.float32),
            pltpu.VMEM((8, 16), jnp.float32),
        ],
        compiler_params=pltpu.CompilerParams(use_tc_tiling_on_sc=True, needs_layout_passes=False),
    )(node_features,
      node_gamma.reshape(8, 16),
      node_beta.reshape(8, 16))

    e_out, t_out = pl.pallas_call(
        _tc_body,
        grid=(_GRID,),
        in_specs=[
            pl.BlockSpec((16, _BL), lambda i: (0, i)),
            pl.BlockSpec((16, _BL), lambda i: (0, i)),
            pl.BlockSpec((8, 16), lambda i: (0, 0)),
            pl.BlockSpec((16, 1), lambda i: (0, 0)),
            pl.BlockSpec((16, 1), lambda i: (0, 0)),
            pl.BlockSpec((16, 1), lambda i: (0, 0)),
            pl.BlockSpec((16, 1), lambda i: (0, 0)),
        ],
        out_specs=[
            pl.BlockSpec((16, _BL), lambda i: (0, i)),
            pl.BlockSpec((16, _BL), lambda i: (0, i)),
        ],
        out_shape=[
            jax.ShapeDtypeStruct((16, _EDGE_ROWS), jnp.float32),
            jax.ShapeDtypeStruct((16, _EDGE_ROWS), jnp.float32),
        ],
        compiler_params=pltpu.CompilerParams(
            dimension_semantics=("arbitrary",)),
    )(e_t, t_t, m8,
      _col16(edge_gamma), _col16(edge_beta),
      _col16(tri_gamma), _col16(tri_beta))

    return (node_out, e_out.T, t_out.T)


# speedup vs baseline: 1.0930x; 1.0930x over previous
import jax
import jax.numpy as jnp
from jax import lax
from jax.experimental import pallas as pl
from jax.experimental.pallas import tpu as pltpu
from jax.experimental.pallas import tpu_sc as plsc

_EPS = 1e-5
_GRID = 10
_NODE_ROWS = 10000
_EDGE_ROWS = 320000
_BL = _EDGE_ROWS // _GRID

_NW = 32
_SC_ROWS = 2000
_TC_ROWS = _NODE_ROWS - _SC_ROWS
_NBLK = _TC_ROWS // _GRID
_CROWS = 40
_NCHUNKS = _SC_ROWS // _CROWS
_JMAX = (_NCHUNKS + _NW - 1) // _NW


def _tc_body(n_ref, e_ref, t_ref, ng_ref, nb_ref, m_ref, eg_ref, eb_ref,
             tg_ref, tb_ref, no_ref, eo_ref, to_ref):
    x = n_ref[...]
    mu = jnp.mean(x, axis=-1, keepdims=True)
    xc = x - mu
    var = jnp.mean(xc * xc, axis=-1, keepdims=True)
    no_ref[...] = xc * jax.lax.rsqrt(var + _EPS) * ng_ref[...] + nb_ref[...]

    row8 = jax.lax.broadcasted_iota(jnp.int32, (8, _BL), 0)
    geom8 = row8 < 3
    m8 = m_ref[...]
    for ref, g_ref, b_ref, o_ref in ((e_ref, eg_ref, eb_ref, eo_ref),
                                     (t_ref, tg_ref, tb_ref, to_ref)):
        x = ref[...]
        s1 = jnp.dot(m8, x, preferred_element_type=jnp.float32)
        s2 = jnp.dot(m8, x * x, preferred_element_type=jnp.float32)
        mu = s1 * (1.0 / 13.0)
        var = s2 * (1.0 / 13.0) - mu * mu
        rstd = jax.lax.rsqrt(var + _EPS)
        g = g_ref[...]
        b = b_ref[...]
        p_lo = rstd * g[0:8, :]
        p_hi = rstd * g[8:16, :]
        q_lo = b[0:8, :] - mu * p_lo
        q_hi = b[8:16, :] - mu * p_hi
        xlo = x[0:8, :]
        xhi = x[8:16, :]
        o_ref[0:8, :] = jnp.where(geom8, xlo, xlo * p_lo + q_lo)
        o_ref[8:16, :] = xhi * p_hi + q_hi


def _sc_rsqrt(v):
    iv = lax.bitcast_convert_type(v, jnp.int32)
    y = lax.bitcast_convert_type(jnp.int32(0x5F3759DF) - (iv >> 1), jnp.float32)
    for _ in range(3):
        y = y * (1.5 - 0.5 * v * y * y)
    return y


def _sc_node_body(n_ref, g_ref, b_ref, o_ref, xbuf, obuf, gbuf, bbuf):
    wid = lax.axis_index("s") * 2 + lax.axis_index("c")
    pltpu.sync_copy(g_ref, gbuf)
    pltpu.sync_copy(b_ref, bbuf)

    def chunk_body(j, carry):
        cid = wid + j * _NW

        @pl.when(cid < _NCHUNKS)
        def _():
            r0 = _TC_ROWS + cid * _CROWS
            pltpu.sync_copy(n_ref.at[pl.ds(r0, _CROWS), :], xbuf)

            def row(k, c2):
                xs = [xbuf[k, pl.ds(l * 16, 16)] for l in range(8)]
                s1v = xs[0]
                for l in range(1, 8):
                    s1v = s1v + xs[l]
                s2v = xs[0] * xs[0]
                for l in range(1, 8):
                    s2v = s2v + xs[l] * xs[l]
                s1 = jnp.sum(s1v)
                s2 = jnp.sum(s2v)
                muv = jnp.full((16,), s1) * (1.0 / 128.0)
                s2m = jnp.full((16,), s2) * (1.0 / 128.0)
                var = s2m - muv * muv
                rstd = _sc_rsqrt(var + _EPS)
                for l in range(8):
                    obuf[k, pl.ds(l * 16, 16)] = ((xs[l] - muv) * rstd
                                                  * gbuf[l, :] + bbuf[l, :])
                return c2

            lax.fori_loop(0, _CROWS, row, 0)
            pltpu.sync_copy(obuf, o_ref.at[pl.ds(cid * _CROWS, _CROWS), :])

        return carry

    lax.fori_loop(0, _JMAX, chunk_body, 0)


def _col16(vec13):
    return jnp.concatenate([jnp.zeros((3,), vec13.dtype), vec13])[:, None]


def kernel(node_features, edge_features, triangle_features,
           node_gamma, node_beta, edge_gamma, edge_beta, tri_gamma, tri_beta):
    e_t = edge_features.T
    t_t = triangle_features.T
    m8 = (jnp.arange(16)[None, :] >= 3).astype(jnp.float32) * jnp.ones((8, 1), jnp.float32)

    mesh = plsc.VectorSubcoreMesh(core_axis_name="c", subcore_axis_name="s")
    node_sc = pl.kernel(
        _sc_node_body,
        out_type=jax.ShapeDtypeStruct((_SC_ROWS, 128), jnp.float32),
        mesh=mesh,
        scratch_types=[
            pltpu.VMEM((_CROWS, 128), jnp.float32),
            pltpu.VMEM((_CROWS, 128), jnp.float32),
            pltpu.VMEM((8, 16), jnp.float32),
            pltpu.VMEM((8, 16), jnp.float32),
        ],
        compiler_params=pltpu.CompilerParams(use_tc_tiling_on_sc=True, needs_layout_passes=False),
    )(node_features,
      node_gamma.reshape(8, 16),
      node_beta.reshape(8, 16))

    node_tc, e_out, t_out = pl.pallas_call(
        _tc_body,
        grid=(_GRID,),
        in_specs=[
            pl.BlockSpec((_NBLK, 128), lambda i: (i, 0)),
            pl.BlockSpec((16, _BL), lambda i: (0, i)),
            pl.BlockSpec((16, _BL), lambda i: (0, i)),
            pl.BlockSpec((1, 128), lambda i: (0, 0)),
            pl.BlockSpec((1, 128), lambda i: (0, 0)),
            pl.BlockSpec((8, 16), lambda i: (0, 0)),
            pl.BlockSpec((16, 1), lambda i: (0, 0)),
            pl.BlockSpec((16, 1), lambda i: (0, 0)),
            pl.BlockSpec((16, 1), lambda i: (0, 0)),
            pl.BlockSpec((16, 1), lambda i: (0, 0)),
        ],
        out_specs=[
            pl.BlockSpec((_NBLK, 128), lambda i: (i, 0)),
            pl.BlockSpec((16, _BL), lambda i: (0, i)),
            pl.BlockSpec((16, _BL), lambda i: (0, i)),
        ],
        out_shape=[
            jax.ShapeDtypeStruct((_TC_ROWS, 128), jnp.float32),
            jax.ShapeDtypeStruct((16, _EDGE_ROWS), jnp.float32),
            jax.ShapeDtypeStruct((16, _EDGE_ROWS), jnp.float32),
        ],
        compiler_params=pltpu.CompilerParams(
            dimension_semantics=("arbitrary",)),
    )(node_features, e_t, t_t,
      node_gamma[None, :], node_beta[None, :], m8,
      _col16(edge_gamma), _col16(edge_beta),
      _col16(tri_gamma), _col16(tri_beta))

    return (jnp.concatenate([node_tc, node_sc], axis=0), e_out.T, t_out.T)


# speedup vs baseline: 1.6006x; 1.4645x over previous
import jax
import jax.numpy as jnp
from jax.experimental import pallas as pl
from jax.experimental.pallas import tpu as pltpu

_EPS = 1e-5
_GRID = 5
_NREV = 1
_NODE_ROWS = 10000
_EDGE_ROWS = 320000
_BL = _EDGE_ROWS // _GRID
_NBLK = _NODE_ROWS // (_GRID // _NREV)


def _ln_body(n_ref, e_ref, t_ref, ng_ref, nb_ref, m_ref, eg_ref, eb_ref,
             tg_ref, tb_ref, no_ref, eo_ref, to_ref):
    x = n_ref[...]
    mu = jnp.mean(x, axis=-1, keepdims=True)
    xc = x - mu
    var = jnp.mean(xc * xc, axis=-1, keepdims=True)
    no_ref[...] = xc * jax.lax.rsqrt(var + _EPS) * ng_ref[...] + nb_ref[...]

    row8 = jax.lax.broadcasted_iota(jnp.int32, (8, _BL), 0)
    geom8 = row8 < 3
    m8 = m_ref[...]
    for ref, g_ref, b_ref, o_ref in ((e_ref, eg_ref, eb_ref, eo_ref),
                                     (t_ref, tg_ref, tb_ref, to_ref)):
        x = ref[...]
        s1 = jnp.dot(m8, x, preferred_element_type=jnp.float32)
        s2 = jnp.dot(m8, x * x, preferred_element_type=jnp.float32)
        mu = s1 * (1.0 / 13.0)
        var = s2 * (1.0 / 13.0) - mu * mu
        rstd = jax.lax.rsqrt(var + _EPS)
        g = g_ref[...]
        b = b_ref[...]
        p_lo = rstd * g[0:8, :]
        p_hi = rstd * g[8:16, :]
        q_lo = b[0:8, :] - mu * p_lo
        q_hi = b[8:16, :] - mu * p_hi
        xlo = x[0:8, :]
        xhi = x[8:16, :]
        o_ref[0:8, :] = jnp.where(geom8, xlo, xlo * p_lo + q_lo)
        o_ref[8:16, :] = xhi * p_hi + q_hi


def _col16(vec13):
    return jnp.concatenate([jnp.zeros((3,), vec13.dtype), vec13])[:, None]


def kernel(node_features, edge_features, triangle_features,
           node_gamma, node_beta, edge_gamma, edge_beta, tri_gamma, tri_beta):
    e_t = edge_features.T
    t_t = triangle_features.T
    m8 = (jnp.arange(16)[None, :] >= 3).astype(jnp.float32) * jnp.ones((8, 1), jnp.float32)

    node_out, e_out, t_out = pl.pallas_call(
        _ln_body,
        grid=(_GRID,),
        in_specs=[
            pl.BlockSpec((_NBLK, 128), lambda i: (i // _NREV, 0)),
            pl.BlockSpec((16, _BL), lambda i: (0, i)),
            pl.BlockSpec((16, _BL), lambda i: (0, i)),
            pl.BlockSpec((1, 128), lambda i: (0, 0)),
            pl.BlockSpec((1, 128), lambda i: (0, 0)),
            pl.BlockSpec((8, 16), lambda i: (0, 0)),
            pl.BlockSpec((16, 1), lambda i: (0, 0)),
            pl.BlockSpec((16, 1), lambda i: (0, 0)),
            pl.BlockSpec((16, 1), lambda i: (0, 0)),
            pl.BlockSpec((16, 1), lambda i: (0, 0)),
        ],
        out_specs=[
            pl.BlockSpec((_NBLK, 128), lambda i: (i // _NREV, 0)),
            pl.BlockSpec((16, _BL), lambda i: (0, i)),
            pl.BlockSpec((16, _BL), lambda i: (0, i)),
        ],
        out_shape=[
            jax.ShapeDtypeStruct((_NODE_ROWS, 128), jnp.float32),
            jax.ShapeDtypeStruct((16, _EDGE_ROWS), jnp.float32),
            jax.ShapeDtypeStruct((16, _EDGE_ROWS), jnp.float32),
        ],
        compiler_params=pltpu.CompilerParams(
            dimension_semantics=("arbitrary",)),
    )(node_features, e_t, t_t,
      node_gamma[None, :], node_beta[None, :], m8,
      _col16(edge_gamma), _col16(edge_beta),
      _col16(tri_gamma), _col16(tri_beta))

    return (node_out, e_out.T, t_out.T)
